# Initial kernel scaffold; baseline (speedup 1.0000x reference)
import jax
import jax.numpy as jnp
from jax.experimental import pallas as pl


def kernel(x):
    raise NotImplementedError("write your pallas kernel here")


# speedup vs baseline: 1.0062x; 1.0062x over previous
import jax
import jax.numpy as jnp
from jax.experimental import pallas as pl

_BLOCK_ROWS = 2048


def _copy_body(x_ref, o_ref):
    o_ref[...] = x_ref[...]


def kernel(x):
    rows, cols = x.shape
    grid = (rows // _BLOCK_ROWS,)
    return pl.pallas_call(
        _copy_body,
        grid=grid,
        in_specs=[pl.BlockSpec((_BLOCK_ROWS, cols), lambda i: (i, 0))],
        out_specs=pl.BlockSpec((_BLOCK_ROWS, cols), lambda i: (i, 0)),
        out_shape=jax.ShapeDtypeStruct((rows, cols), x.dtype),
    )(x)
